# baseline (speedup 1.0000x reference)
import math
import functools

import jax
import jax.numpy as jnp
from jax import lax
from jax.experimental import pallas as pl
from jax.experimental.pallas import tpu as pltpu


def _attn_kernel(cen_ref, wq_ref, wkv_ref, wo_ref, y_ref, st_ref, *,
                 gblk, num_heads, hidden, hid8, H, W, inv_sqrt_area):
    hslice = hid8 // num_heads

    def _roll(x, s):
        return jnp.roll(x, s, axis=1)

    cens = [cen_ref[g] for g in range(gblk)]
    pix = lax.broadcasted_iota(jnp.int32, cens[0].shape, 1)
    row = lax.shift_right_logical(pix, 5)
    col = lax.bitwise_and(pix, W - 1)

    cens_bf = [c.astype(jnp.bfloat16) for c in cens]
    q_alls = [jnp.dot(wq_ref[...], c,
                      preferred_element_type=jnp.float32) for c in cens_bf]

    kbfs = [[None] * num_heads for _ in range(gblk)]
    vbfs = [[None] * num_heads for _ in range(gblk)]
    for h in range(num_heads):
        d = (1, 2)[h]
        for g in range(gblk):
            cen = cens[g]
            if d == 1:
                rneg = lambda x: jnp.where(row == 0, _roll(x, -W),
                                           _roll(x, W))
                rpos = lambda x: jnp.where(row == H - 1, _roll(x, W),
                                           _roll(x, -W))
                cn = jnp.where(col == 0, _roll(cen, -1), _roll(cen, 1))
                cp = jnp.where(col == W - 1, _roll(cen, 1), _roll(cen, -1))
            else:
                rneg = lambda x: jnp.where(
                    row == 0, _roll(x, -2 * W),
                    jnp.where(row == 1, x, _roll(x, 2 * W)))
                rpos = lambda x: jnp.where(
                    row == H - 2, x,
                    jnp.where(row == H - 1, _roll(x, 2 * W),
                              _roll(x, -2 * W)))
                cn = jnp.where(col == 0, _roll(cen, -2),
                               jnp.where(col == 1, cen, _roll(cen, 2)))
                cp = jnp.where(col == W - 2, cen,
                               jnp.where(col == W - 1, _roll(cen, 2),
                                         _roll(cen, -2)))
            imgs = (rneg(cn), rneg(cen), rneg(cp), cn, cp,
                    rpos(cn), rpos(cen), rpos(cp))
            cbf = cens_bf[g]
            sur = jnp.concatenate(
                [im.astype(jnp.bfloat16) - cbf for im in imgs],
                axis=0)
            kv = jnp.dot(wkv_ref[h], sur,
                         preferred_element_type=jnp.float32)
            k = kv[:hid8]
            kn = k * lax.rsqrt(jnp.maximum(
                jnp.sum(k * k, axis=-1, keepdims=True), 1e-24))
            kbfs[g][h] = kn.astype(jnp.bfloat16)
            vbfs[g][h] = kv[hid8:].astype(jnp.bfloat16)

    cnt = hidden * hid8
    ss, vsel_ = [], []
    for g in range(gblk):
        for n in range(num_heads):
            lo = n * hslice
            kn = jnp.concatenate([kb[lo:lo + hslice] for kb in kbfs[g]],
                                 axis=0)
            v = jnp.concatenate([vb[lo:lo + hslice] for vb in vbfs[g]],
                                axis=0)
            q = q_alls[g][n * hidden:(n + 1) * hidden]
            qn = (q * (lax.rsqrt(jnp.maximum(
                jnp.sum(q * q, axis=-1, keepdims=True), 1e-24))
                * inv_sqrt_area)).astype(jnp.bfloat16)
            ss.append(lax.dot_general(qn, kn, (((1,), (1,)), ((), ())),
                                      preferred_element_type=jnp.float32))
            vsel_.append(v)

    ps = []
    for s in ss:
        tot = jnp.sum(jnp.sum(s, axis=-1, keepdims=True),
                      axis=0, keepdims=True)
        tot2 = jnp.sum(jnp.sum(s * s, axis=-1, keepdims=True),
                       axis=0, keepdims=True)
        mu = tot / cnt
        var = tot2 / cnt - mu * mu
        e = jnp.exp((s - mu) * lax.rsqrt(var + 1e-5))
        ps.append((e / jnp.sum(e, axis=-1, keepdims=True)
                   ).astype(jnp.bfloat16))

    for g in range(gblk):
        wps = [jnp.dot(wo_ref[n], ps[g * num_heads + n],
                       preferred_element_type=jnp.float32
                       ).astype(jnp.bfloat16)
               for n in range(num_heads)]
        wp_all = jnp.concatenate(wps, axis=1)
        v_all = jnp.concatenate(
            [vsel_[g * num_heads + n] for n in range(num_heads)], axis=0)
        y = jnp.dot(wp_all, v_all, preferred_element_type=jnp.float32)
        y_ref[g] = y.astype(jnp.bfloat16)
        st_ref[g] = jnp.concatenate(
            [jnp.sum(y, axis=1, keepdims=True),
             jnp.sum(y * y, axis=1, keepdims=True)], axis=1)


def _bn_relu_kernel(y_ref, st_ref, o_ref, *, count):
    tot = jnp.sum(st_ref[...], axis=0)
    inv = 1.0 / count
    mu = tot[:, 0:1] * inv
    var = tot[:, 1:2] * inv - mu * mu
    scale = lax.rsqrt(var + 1e-5)
    o_ref[...] = jnp.maximum(
        (y_ref[...].astype(jnp.float32) - mu) * scale, 0.0)


def kernel(wq, wk, wv, wo, cen):
    B, C, H, W = cen.shape
    NH, hidden = wq.shape[0], wq.shape[1]
    hid8 = wk.shape[1]
    tra = NH * hidden
    out_ch = wo.shape[0]
    HW = H * W

    cen_flat = cen.astype(jnp.float32).reshape(B, C, HW)

    wq_perm = wq.transpose(1, 0, 2).reshape(tra, C).astype(jnp.bfloat16)
    wkv = jnp.stack([jnp.concatenate([wk[h], wv[h]], axis=0)
                     for h in range(NH)]).astype(jnp.bfloat16)

    gblk = min(4, B)
    attn = functools.partial(
        _attn_kernel, gblk=gblk, num_heads=NH, hidden=hidden, hid8=hid8,
        H=H, W=W, inv_sqrt_area=1.0 / math.sqrt(HW))

    wo_r = wo.reshape(out_ch, NH, hidden).transpose(1, 0, 2).astype(
        jnp.bfloat16)

    y_pre, stats = pl.pallas_call(
        attn,
        out_shape=(jax.ShapeDtypeStruct((B, out_ch, HW), jnp.bfloat16),
                   jax.ShapeDtypeStruct((B, out_ch, 2), jnp.float32)),
        grid=(B // gblk,),
        in_specs=[
            pl.BlockSpec((gblk, C, HW), lambda b: (b, 0, 0)),
            pl.BlockSpec((tra, C), lambda b: (0, 0)),
            pl.BlockSpec((NH, 2 * hid8, 8 * C), lambda b: (0, 0, 0)),
            pl.BlockSpec((NH, out_ch, hidden), lambda b: (0, 0, 0)),
        ],
        out_specs=(pl.BlockSpec((gblk, out_ch, HW), lambda b: (b, 0, 0)),
                   pl.BlockSpec((gblk, out_ch, 2), lambda b: (b, 0, 0))),
        compiler_params=pltpu.CompilerParams(
            dimension_semantics=("parallel",)),
    )(cen_flat, wq_perm, wkv, wo_r)

    bblk = min(32, B)
    bn = functools.partial(_bn_relu_kernel, count=float(B * HW))
    y = pl.pallas_call(
        bn,
        out_shape=jax.ShapeDtypeStruct((B, out_ch, HW), jnp.float32),
        grid=(B // bblk,),
        in_specs=[
            pl.BlockSpec((bblk, out_ch, HW), lambda b: (b, 0, 0)),
            pl.BlockSpec((B, out_ch, 2), lambda b: (0, 0, 0)),
        ],
        out_specs=pl.BlockSpec((bblk, out_ch, HW), lambda b: (b, 0, 0)),
        compiler_params=pltpu.CompilerParams(
            dimension_semantics=("parallel",)),
    )(y_pre, stats)

    return y.reshape(B, out_ch, H, W)


# speedup vs baseline: 1.0241x; 1.0241x over previous
import math
import functools

import jax
import jax.numpy as jnp
from jax import lax
from jax.experimental import pallas as pl
from jax.experimental.pallas import tpu as pltpu


def _attn_kernel(cen_ref, wq_ref, wkv_ref, wo_ref, y_ref, st_ref, *,
                 gblk, num_heads, hidden, hid8, H, W, inv_sqrt_area):
    hslice = hid8 // num_heads

    def _roll(x, s):
        return jnp.roll(x, s, axis=1)

    cens = [cen_ref[g] for g in range(gblk)]
    pix = lax.broadcasted_iota(jnp.int32, cens[0].shape, 1)
    row = lax.shift_right_logical(pix, 5)
    col = lax.bitwise_and(pix, W - 1)

    cens_bf = [c.astype(jnp.bfloat16) for c in cens]
    q_alls = [jnp.dot(wq_ref[...], c,
                      preferred_element_type=jnp.float32) for c in cens_bf]

    kbfs = [[None] * num_heads for _ in range(gblk)]
    vbfs = [[None] * num_heads for _ in range(gblk)]
    for h in range(num_heads):
        d = (1, 2)[h]
        for g in range(gblk):
            cen = cens[g]
            if d == 1:
                rneg = lambda x: jnp.where(row == 0, _roll(x, -W),
                                           _roll(x, W))
                rpos = lambda x: jnp.where(row == H - 1, _roll(x, W),
                                           _roll(x, -W))
                cn = jnp.where(col == 0, _roll(cen, -1), _roll(cen, 1))
                cp = jnp.where(col == W - 1, _roll(cen, 1), _roll(cen, -1))
            else:
                rneg = lambda x: jnp.where(
                    row == 0, _roll(x, -2 * W),
                    jnp.where(row == 1, x, _roll(x, 2 * W)))
                rpos = lambda x: jnp.where(
                    row == H - 2, x,
                    jnp.where(row == H - 1, _roll(x, 2 * W),
                              _roll(x, -2 * W)))
                cn = jnp.where(col == 0, _roll(cen, -2),
                               jnp.where(col == 1, cen, _roll(cen, 2)))
                cp = jnp.where(col == W - 2, cen,
                               jnp.where(col == W - 1, _roll(cen, 2),
                                         _roll(cen, -2)))
            imgs = (rneg(cn), rneg(cen), rneg(cp), cn, cp,
                    rpos(cn), rpos(cen), rpos(cp))
            sur = jnp.concatenate([im - cen for im in imgs],
                                  axis=0).astype(jnp.bfloat16)
            kv = jnp.dot(wkv_ref[h], sur,
                         preferred_element_type=jnp.float32)
            k = kv[:hid8]
            kn = k * lax.rsqrt(jnp.maximum(
                jnp.sum(k * k, axis=-1, keepdims=True), 1e-24))
            kbfs[g][h] = kn.astype(jnp.bfloat16)
            vbfs[g][h] = kv[hid8:].astype(jnp.bfloat16)

    cnt = hidden * hid8
    ss, vsel_ = [], []
    for g in range(gblk):
        for n in range(num_heads):
            lo = n * hslice
            kn = jnp.concatenate([kb[lo:lo + hslice] for kb in kbfs[g]],
                                 axis=0)
            v = jnp.concatenate([vb[lo:lo + hslice] for vb in vbfs[g]],
                                axis=0)
            q = q_alls[g][n * hidden:(n + 1) * hidden]
            qn = (q * (lax.rsqrt(jnp.maximum(
                jnp.sum(q * q, axis=-1, keepdims=True), 1e-24))
                * inv_sqrt_area)).astype(jnp.bfloat16)
            ss.append(lax.dot_general(qn, kn, (((1,), (1,)), ((), ())),
                                      preferred_element_type=jnp.float32))
            vsel_.append(v)

    ps = []
    for s in ss:
        tot = jnp.sum(jnp.sum(s, axis=-1, keepdims=True),
                      axis=0, keepdims=True)
        tot2 = jnp.sum(jnp.sum(s * s, axis=-1, keepdims=True),
                       axis=0, keepdims=True)
        mu = tot / cnt
        var = tot2 / cnt - mu * mu
        e = jnp.exp((s - mu) * lax.rsqrt(var + 1e-5))
        ps.append((e / jnp.sum(e, axis=-1, keepdims=True)
                   ).astype(jnp.bfloat16))

    for g in range(gblk):
        wps = [jnp.dot(wo_ref[n], ps[g * num_heads + n],
                       preferred_element_type=jnp.float32
                       ).astype(jnp.bfloat16)
               for n in range(num_heads)]
        wp_all = jnp.concatenate(wps, axis=1)
        v_all = jnp.concatenate(
            [vsel_[g * num_heads + n] for n in range(num_heads)], axis=0)
        y = jnp.dot(wp_all, v_all, preferred_element_type=jnp.float32)
        y_ref[g] = y.astype(jnp.bfloat16)
        st_ref[g] = jnp.concatenate(
            [jnp.sum(y, axis=1, keepdims=True),
             jnp.sum(y * y, axis=1, keepdims=True)], axis=1)


def _bn_relu_kernel(y_ref, st_ref, o_ref, *, count):
    tot = jnp.sum(st_ref[...], axis=0)
    inv = 1.0 / count
    mu = tot[:, 0:1] * inv
    var = tot[:, 1:2] * inv - mu * mu
    scale = lax.rsqrt(var + 1e-5)
    o_ref[...] = jnp.maximum(
        (y_ref[...].astype(jnp.float32) - mu) * scale, 0.0)


def kernel(wq, wk, wv, wo, cen):
    B, C, H, W = cen.shape
    NH, hidden = wq.shape[0], wq.shape[1]
    hid8 = wk.shape[1]
    tra = NH * hidden
    out_ch = wo.shape[0]
    HW = H * W

    cen_flat = cen.astype(jnp.float32).reshape(B, C, HW)

    wq_perm = wq.transpose(1, 0, 2).reshape(tra, C).astype(jnp.bfloat16)
    wkv = jnp.stack([jnp.concatenate([wk[h], wv[h]], axis=0)
                     for h in range(NH)]).astype(jnp.bfloat16)

    gblk = min(4, B)
    attn = functools.partial(
        _attn_kernel, gblk=gblk, num_heads=NH, hidden=hidden, hid8=hid8,
        H=H, W=W, inv_sqrt_area=1.0 / math.sqrt(HW))

    wo_r = wo.reshape(out_ch, NH, hidden).transpose(1, 0, 2).astype(
        jnp.bfloat16)

    y_pre, stats = pl.pallas_call(
        attn,
        out_shape=(jax.ShapeDtypeStruct((B, out_ch, HW), jnp.bfloat16),
                   jax.ShapeDtypeStruct((B, out_ch, 2), jnp.float32)),
        grid=(B // gblk,),
        in_specs=[
            pl.BlockSpec((gblk, C, HW), lambda b: (b, 0, 0)),
            pl.BlockSpec((tra, C), lambda b: (0, 0)),
            pl.BlockSpec((NH, 2 * hid8, 8 * C), lambda b: (0, 0, 0)),
            pl.BlockSpec((NH, out_ch, hidden), lambda b: (0, 0, 0)),
        ],
        out_specs=(pl.BlockSpec((gblk, out_ch, HW), lambda b: (b, 0, 0)),
                   pl.BlockSpec((gblk, out_ch, 2), lambda b: (b, 0, 0))),
        compiler_params=pltpu.CompilerParams(
            dimension_semantics=("parallel",)),
    )(cen_flat, wq_perm, wkv, wo_r)

    bblk = min(32, B)
    bn = functools.partial(_bn_relu_kernel, count=float(B * HW))
    y = pl.pallas_call(
        bn,
        out_shape=jax.ShapeDtypeStruct((B, out_ch, HW), jnp.float32),
        grid=(B // bblk,),
        in_specs=[
            pl.BlockSpec((bblk, out_ch, HW), lambda b: (b, 0, 0)),
            pl.BlockSpec((B, out_ch, 2), lambda b: (0, 0, 0)),
        ],
        out_specs=pl.BlockSpec((bblk, out_ch, HW), lambda b: (b, 0, 0)),
        compiler_params=pltpu.CompilerParams(
            dimension_semantics=("parallel",)),
    )(y_pre, stats)

    return y.reshape(B, out_ch, H, W)
